# Initial kernel scaffold; baseline (speedup 1.0000x reference)
import jax
import jax.numpy as jnp
from jax import lax
from jax.experimental import pallas as pl
from jax.experimental.pallas import tpu as pltpu
from jax.experimental.pallas import tpu_sc as plsc

N = 10000
E = 320000
DN = 128
DE = 16
HID = 128

NC, NS = 2, 16
NW = NC * NS

F32 = jnp.float32

_MESH = plsc.VectorSubcoreMesh(
    core_axis_name="c", subcore_axis_name="s", num_cores=NC, num_subcores=NS)

GROUP = 80
GCHUNK = 5
GB = GROUP * GCHUNK
G_ROWS_T = E // NW // GROUP
G_ITERS = G_ROWS_T // GCHUNK


def _gather_body(ps_hbm, pr_hbm, sidx_hbm, ridx_hbm, gs_hbm, gr_hbm,
                 idx_v, buf_v, sem):
    cid = lax.axis_index("c")
    sid = lax.axis_index("s")
    wid = sid * NC + cid
    row0 = wid * G_ROWS_T

    def chunk(c, carry):
        r0 = row0 + c * GCHUNK
        base = r0 * GROUP
        pltpu.sync_copy(sidx_hbm.at[pl.ds(r0, GCHUNK)], idx_v)
        descs = [
            pltpu.async_copy(ps_hbm.at[idx_v.at[j]],
                             buf_v.at[pl.ds(j * GROUP, GROUP)], sem)
            for j in range(GCHUNK)
        ]
        for d in descs:
            d.wait()
        pltpu.sync_copy(buf_v, gs_hbm.at[pl.ds(base, GB)])

        pltpu.sync_copy(ridx_hbm.at[pl.ds(r0, GCHUNK)], idx_v)
        descs = [
            pltpu.async_copy(pr_hbm.at[idx_v.at[j]],
                             buf_v.at[pl.ds(j * GROUP, GROUP)], sem)
            for j in range(GCHUNK)
        ]
        for d in descs:
            d.wait()
        pltpu.sync_copy(buf_v, gr_hbm.at[pl.ds(base, GB)])
        return carry

    lax.fori_loop(0, G_ITERS, chunk, 0)


_gather = pl.kernel(
    _gather_body,
    out_type=(jax.ShapeDtypeStruct((E, DN), F32),
              jax.ShapeDtypeStruct((E, DN), F32)),
    mesh=_MESH,
    scratch_types=[
        pltpu.VMEM((GCHUNK, GROUP), jnp.int32),
        pltpu.VMEM((GB, DN), F32),
        pltpu.SemaphoreType.DMA,
    ],
)

VW = 32
SGROUP = 80
SCHUNK = 5
SB = SGROUP * SCHUNK
ROWS_T = 2 * E // NW
S_ROWS_T = ROWS_T // SGROUP
S_ITERS = S_ROWS_T // SCHUNK
ASTRIPE = 2 * N // NS


def _scatter_body(v_hbm, idx_hbm, out_hbm, idx_v, val_v, stripe_v, a_sh, sem):
    cid = lax.axis_index("c")
    sid = lax.axis_index("s")
    wid = sid * NC + cid

    z = jnp.zeros((16,), F32)

    def zrow(i, carry):
        stripe_v[i, pl.ds(0, 16)] = z
        stripe_v[i, pl.ds(16, 16)] = z
        return carry

    lax.fori_loop(0, ASTRIPE, zrow, 0)
    pltpu.sync_copy(stripe_v, a_sh.at[pl.ds(sid * ASTRIPE, ASTRIPE)])
    plsc.subcore_barrier()

    row0 = wid * S_ROWS_T

    def chunk(c, carry):
        r0 = row0 + c * SCHUNK
        base = r0 * SGROUP
        pltpu.sync_copy(idx_hbm.at[pl.ds(r0, SCHUNK)], idx_v)
        pltpu.sync_copy(v_hbm.at[pl.ds(base, SB)], val_v)
        for j in range(SCHUNK):
            pltpu.sync_copy(val_v.at[pl.ds(j * SGROUP, SGROUP)],
                            a_sh.at[idx_v.at[j]], add=True)
        return carry

    lax.fori_loop(0, S_ITERS, chunk, 0)
    plsc.subcore_barrier()

    pltpu.sync_copy(a_sh.at[pl.ds(sid * ASTRIPE, ASTRIPE)], stripe_v)
    pltpu.sync_copy(stripe_v, out_hbm.at[cid, pl.ds(sid * ASTRIPE, ASTRIPE)])


_scatter = pl.kernel(
    _scatter_body,
    out_type=jax.ShapeDtypeStruct((NC, 2 * N, VW), F32),
    mesh=_MESH,
    scratch_types=[
        pltpu.VMEM((SCHUNK, SGROUP), jnp.int32),
        pltpu.VMEM((SB, VW), F32),
        pltpu.VMEM((ASTRIPE, VW), F32),
        pltpu.VMEM_SHARED((2 * N, VW), F32),
        pltpu.SemaphoreType.DMA,
    ],
)

BN1 = 2000


def _proj_body(nodes_ref, ws_ref, wr_ref, ps_ref, pr_ref):
    x = nodes_ref[...]
    ps_ref[...] = jnp.dot(x, ws_ref[...], preferred_element_type=F32)
    pr_ref[...] = jnp.dot(x, wr_ref[...], preferred_element_type=F32)


_proj = pl.pallas_call(
    _proj_body,
    grid=(N // BN1,),
    in_specs=[
        pl.BlockSpec((BN1, DN), lambda i: (i, 0)),
        pl.BlockSpec((DN, DN), lambda i: (0, 0)),
        pl.BlockSpec((DN, DN), lambda i: (0, 0)),
    ],
    out_specs=(pl.BlockSpec((BN1, DN), lambda i: (i, 0)),
               pl.BlockSpec((BN1, DN), lambda i: (i, 0))),
    out_shape=(jax.ShapeDtypeStruct((N, DN), F32),
               jax.ShapeDtypeStruct((N, DN), F32)),
)

BE = 2000


def _edge_body(edges_ref, gs_ref, gr_ref, we_ref, eb1_ref, ew2_ref, eb2_ref,
               eg_ref, ebeta_ref, wsr_ref, bsr_ref, eo_ref, v_ref):
    ed = edges_ref[...]
    h = (jnp.dot(ed, we_ref[...], preferred_element_type=F32)
         + gs_ref[...] + gr_ref[...] + eb1_ref[...])
    h = jnp.maximum(h, 0.0)
    o = jnp.dot(h, ew2_ref[...], preferred_element_type=F32) + eb2_ref[...]
    mu = jnp.mean(o, axis=-1, keepdims=True)
    var = jnp.mean((o - mu) ** 2, axis=-1, keepdims=True)
    u = (o - mu) * lax.rsqrt(var + 1e-5) * eg_ref[...] + ebeta_ref[...]
    eo_ref[...] = ed + u
    lg = jnp.dot(ed, wsr_ref[...], preferred_element_type=F32) + bsr_ref[...]
    ee = jnp.exp(lg)
    es = ee[:, 0:1]
    er = ee[:, 1:2]
    pad = jnp.zeros((BE, VW - DE - 1), F32)
    v_ref[0] = jnp.concatenate([u * es, es, pad], axis=1)
    v_ref[1] = jnp.concatenate([u * er, er, pad], axis=1)


_edge = pl.pallas_call(
    _edge_body,
    grid=(E // BE,),
    in_specs=[
        pl.BlockSpec((BE, DE), lambda i: (i, 0)),
        pl.BlockSpec((BE, DN), lambda i: (i, 0)),
        pl.BlockSpec((BE, DN), lambda i: (i, 0)),
        pl.BlockSpec((DE, DN), lambda i: (0, 0)),
        pl.BlockSpec((1, DN), lambda i: (0, 0)),
        pl.BlockSpec((DN, DE), lambda i: (0, 0)),
        pl.BlockSpec((1, DE), lambda i: (0, 0)),
        pl.BlockSpec((1, DE), lambda i: (0, 0)),
        pl.BlockSpec((1, DE), lambda i: (0, 0)),
        pl.BlockSpec((DE, 2), lambda i: (0, 0)),
        pl.BlockSpec((1, 2), lambda i: (0, 0)),
    ],
    out_specs=(pl.BlockSpec((BE, DE), lambda i: (i, 0)),
               pl.BlockSpec((2, BE, VW), lambda i: (0, i, 0))),
    out_shape=(jax.ShapeDtypeStruct((E, DE), F32),
               jax.ShapeDtypeStruct((2, E, VW), F32)),
)

BN = 2000


def _node_body(nodes_ref, as_ref, ar_ref, w1n_ref, w1r_ref, w1s_ref, nb1_ref,
               nw2_ref, nb2_ref, ng_ref, nbeta_ref, out_ref):
    x = nodes_ref[...]
    a_s = as_ref[0] + as_ref[1]
    a_r = ar_ref[0] + ar_ref[1]
    ss = a_s[:, DE:DE + 1]
    sr = a_r[:, DE:DE + 1]
    sent = jnp.where(ss > 0, a_s[:, :DE] / jnp.where(ss > 0, ss, 1.0), 0.0)
    recv = jnp.where(sr > 0, a_r[:, :DE] / jnp.where(sr > 0, sr, 1.0), 0.0)
    h = (jnp.dot(x, w1n_ref[...], preferred_element_type=F32)
         + jnp.dot(recv, w1r_ref[...], preferred_element_type=F32)
         + jnp.dot(sent, w1s_ref[...], preferred_element_type=F32)
         + nb1_ref[...])
    h = jnp.maximum(h, 0.0)
    o = jnp.dot(h, nw2_ref[...], preferred_element_type=F32) + nb2_ref[...]
    mu = jnp.mean(o, axis=-1, keepdims=True)
    var = jnp.mean((o - mu) ** 2, axis=-1, keepdims=True)
    out_ref[...] = x + ((o - mu) * lax.rsqrt(var + 1e-5) * ng_ref[...]
                        + nbeta_ref[...])


_node = pl.pallas_call(
    _node_body,
    grid=(N // BN,),
    in_specs=[
        pl.BlockSpec((BN, DN), lambda i: (i, 0)),
        pl.BlockSpec((NC, BN, VW), lambda i: (0, i, 0)),
        pl.BlockSpec((NC, BN, VW), lambda i: (0, i + N // BN, 0)),
        pl.BlockSpec((DN, DN), lambda i: (0, 0)),
        pl.BlockSpec((DE, DN), lambda i: (0, 0)),
        pl.BlockSpec((DE, DN), lambda i: (0, 0)),
        pl.BlockSpec((1, DN), lambda i: (0, 0)),
        pl.BlockSpec((DN, DN), lambda i: (0, 0)),
        pl.BlockSpec((1, DN), lambda i: (0, 0)),
        pl.BlockSpec((1, DN), lambda i: (0, 0)),
        pl.BlockSpec((1, DN), lambda i: (0, 0)),
    ],
    out_specs=pl.BlockSpec((BN, DN), lambda i: (i, 0)),
    out_shape=jax.ShapeDtypeStruct((N, DN), F32),
)


def kernel(nodes, edges, senders, receivers,
           eW1, eb1, eW2, eb2, eg, ebeta,
           nW1, nb1, nW2, nb2, ng, nbeta,
           rW, rb, sW, sb):
    we = eW1[:DE]
    ws = eW1[DE:DE + DN]
    wr = eW1[DE + DN:]

    ps, pr = _proj(nodes, ws, wr)

    sidx = senders.reshape(E // GROUP, GROUP)
    ridx = receivers.reshape(E // GROUP, GROUP)
    gs, gr = _gather(ps, pr, sidx, ridx)

    wsr = jnp.concatenate([sW, rW], axis=1)
    bsr = jnp.concatenate([sb, rb]).reshape(1, 2)
    edges_out, v = _edge(edges, gs, gr, we, eb1.reshape(1, HID), eW2,
                         eb2.reshape(1, DE), eg.reshape(1, DE),
                         ebeta.reshape(1, DE), wsr, bsr)

    vflat = v.reshape(2 * E, VW)
    idx_comb = jnp.concatenate([senders, receivers + N]).reshape(
        2 * E // SGROUP, SGROUP)
    a = _scatter(vflat, idx_comb)

    nodes_out = _node(nodes, a, a, nW1[:DN], nW1[DN:DN + DE], nW1[DN + DE:],
                      nb1.reshape(1, HID), nW2, nb2.reshape(1, DN),
                      ng.reshape(1, DN), nbeta.reshape(1, DN))
    return nodes_out, edges_out


# speedup vs baseline: 9.5422x; 9.5422x over previous
import jax
import jax.numpy as jnp
from jax import lax
from jax.experimental import pallas as pl
from jax.experimental.pallas import tpu as pltpu
from jax.experimental.pallas import tpu_sc as plsc

N = 10000
E = 320000
DN = 128
DE = 16
HID = 128

NC, NS = 2, 16
NW = NC * NS

F32 = jnp.float32

_MESH = plsc.VectorSubcoreMesh(
    core_axis_name="c", subcore_axis_name="s", num_cores=NC, num_subcores=NS)

GROUP = 80
GCHUNK = 5
GB = GROUP * GCHUNK
G_EDGES_T = E // NW
G_ITERS = G_EDGES_T // GB


def _gather_body(ps_hbm, pr_hbm, sidx_hbm, ridx_hbm, gs_hbm, gr_hbm,
                 idx_v, buf_v, sem):
    cid = lax.axis_index("c")
    sid = lax.axis_index("s")
    wid = sid * NC + cid
    e0 = wid * G_EDGES_T

    def chunk(c, carry):
        base = e0 + c * GB
        pltpu.sync_copy(sidx_hbm.at[pl.ds(base, GB)], idx_v)
        descs = [
            pltpu.async_copy(ps_hbm.at[idx_v.at[pl.ds(j * GROUP, GROUP)]],
                             buf_v.at[pl.ds(j * GROUP, GROUP)], sem)
            for j in range(GCHUNK)
        ]
        for d in descs:
            d.wait()
        pltpu.sync_copy(buf_v, gs_hbm.at[pl.ds(base, GB)])

        pltpu.sync_copy(ridx_hbm.at[pl.ds(base, GB)], idx_v)
        descs = [
            pltpu.async_copy(pr_hbm.at[idx_v.at[pl.ds(j * GROUP, GROUP)]],
                             buf_v.at[pl.ds(j * GROUP, GROUP)], sem)
            for j in range(GCHUNK)
        ]
        for d in descs:
            d.wait()
        pltpu.sync_copy(buf_v, gr_hbm.at[pl.ds(base, GB)])
        return carry

    lax.fori_loop(0, G_ITERS, chunk, 0)


_gather = pl.kernel(
    _gather_body,
    out_type=(jax.ShapeDtypeStruct((E, DN), F32),
              jax.ShapeDtypeStruct((E, DN), F32)),
    mesh=_MESH,
    scratch_types=[
        pltpu.VMEM((GB,), jnp.int32),
        pltpu.VMEM((GB, DN), F32),
        pltpu.SemaphoreType.DMA,
    ],
)

VW = 32
SROW = 128
S_NROWS = 2 * E // SROW
S_BASE = S_NROWS // NW
S_XTRA = S_NROWS - S_BASE * NW
ASTRIPE = 2 * N // NS


def _scatter_body(v_hbm, idx_hbm, out_hbm, idx_v, val_v, stripe_v, a_sh, sem):
    cid = lax.axis_index("c")
    sid = lax.axis_index("s")
    wid = sid * NC + cid

    z = jnp.zeros((16,), F32)

    def zrow(i, carry):
        stripe_v[i, pl.ds(0, 16)] = z
        stripe_v[i, pl.ds(16, 16)] = z
        return carry

    lax.fori_loop(0, ASTRIPE, zrow, 0)
    pltpu.sync_copy(stripe_v, a_sh.at[pl.ds(sid * ASTRIPE, ASTRIPE)])
    plsc.subcore_barrier()

    row0 = S_BASE * wid + jnp.minimum(wid, S_XTRA)
    nrows = S_BASE + (wid < S_XTRA).astype(jnp.int32)

    def chunk(c, carry):
        r = row0 + c
        pltpu.sync_copy(idx_hbm.at[r], idx_v)
        pltpu.sync_copy(v_hbm.at[pl.ds(r * SROW, SROW)], val_v)
        pltpu.sync_copy(val_v, a_sh.at[idx_v], add=True)
        return carry

    lax.fori_loop(0, nrows, chunk, 0)
    plsc.subcore_barrier()

    pltpu.sync_copy(a_sh.at[pl.ds(sid * ASTRIPE, ASTRIPE)], stripe_v)
    pltpu.sync_copy(stripe_v, out_hbm.at[cid, pl.ds(sid * ASTRIPE, ASTRIPE)])


_scatter = pl.kernel(
    _scatter_body,
    out_type=jax.ShapeDtypeStruct((NC, 2 * N, VW), F32),
    mesh=_MESH,
    scratch_types=[
        pltpu.VMEM((SROW,), jnp.int32),
        pltpu.VMEM((SROW, VW), F32),
        pltpu.VMEM((ASTRIPE, VW), F32),
        pltpu.VMEM_SHARED((2 * N, VW), F32),
        pltpu.SemaphoreType.DMA,
    ],
    compiler_params=pltpu.CompilerParams(use_tc_tiling_on_sc=False),
)

BN1 = 2000


def _proj_body(nodes_ref, ws_ref, wr_ref, ps_ref, pr_ref):
    x = nodes_ref[...]
    ps_ref[...] = jnp.dot(x, ws_ref[...], preferred_element_type=F32)
    pr_ref[...] = jnp.dot(x, wr_ref[...], preferred_element_type=F32)


_proj = pl.pallas_call(
    _proj_body,
    grid=(N // BN1,),
    in_specs=[
        pl.BlockSpec((BN1, DN), lambda i: (i, 0)),
        pl.BlockSpec((DN, DN), lambda i: (0, 0)),
        pl.BlockSpec((DN, DN), lambda i: (0, 0)),
    ],
    out_specs=(pl.BlockSpec((BN1, DN), lambda i: (i, 0)),
               pl.BlockSpec((BN1, DN), lambda i: (i, 0))),
    out_shape=(jax.ShapeDtypeStruct((N, DN), F32),
               jax.ShapeDtypeStruct((N, DN), F32)),
)

BE = 2000


def _edge_body(edges_ref, gs_ref, gr_ref, we_ref, eb1_ref, ew2_ref, eb2_ref,
               eg_ref, ebeta_ref, wsr_ref, bsr_ref, eo_ref, v_ref):
    ed = edges_ref[...]
    h = (jnp.dot(ed, we_ref[...], preferred_element_type=F32)
         + gs_ref[...] + gr_ref[...] + eb1_ref[...])
    h = jnp.maximum(h, 0.0)
    o = jnp.dot(h, ew2_ref[...], preferred_element_type=F32) + eb2_ref[...]
    mu = jnp.mean(o, axis=-1, keepdims=True)
    var = jnp.mean((o - mu) ** 2, axis=-1, keepdims=True)
    u = (o - mu) * lax.rsqrt(var + 1e-5) * eg_ref[...] + ebeta_ref[...]
    eo_ref[...] = ed + u
    lg = jnp.dot(ed, wsr_ref[...], preferred_element_type=F32) + bsr_ref[...]
    ee = jnp.exp(lg)
    es = ee[:, 0:1]
    er = ee[:, 1:2]
    pad = jnp.zeros((BE, VW - DE - 1), F32)
    v_ref[0] = jnp.concatenate([u * es, es, pad], axis=1)
    v_ref[1] = jnp.concatenate([u * er, er, pad], axis=1)


_edge = pl.pallas_call(
    _edge_body,
    grid=(E // BE,),
    in_specs=[
        pl.BlockSpec((BE, DE), lambda i: (i, 0)),
        pl.BlockSpec((BE, DN), lambda i: (i, 0)),
        pl.BlockSpec((BE, DN), lambda i: (i, 0)),
        pl.BlockSpec((DE, DN), lambda i: (0, 0)),
        pl.BlockSpec((1, DN), lambda i: (0, 0)),
        pl.BlockSpec((DN, DE), lambda i: (0, 0)),
        pl.BlockSpec((1, DE), lambda i: (0, 0)),
        pl.BlockSpec((1, DE), lambda i: (0, 0)),
        pl.BlockSpec((1, DE), lambda i: (0, 0)),
        pl.BlockSpec((DE, 2), lambda i: (0, 0)),
        pl.BlockSpec((1, 2), lambda i: (0, 0)),
    ],
    out_specs=(pl.BlockSpec((BE, DE), lambda i: (i, 0)),
               pl.BlockSpec((2, BE, VW), lambda i: (0, i, 0))),
    out_shape=(jax.ShapeDtypeStruct((E, DE), F32),
               jax.ShapeDtypeStruct((2, E, VW), F32)),
)

BN = 2000


def _node_body(nodes_ref, as_ref, ar_ref, w1n_ref, w1r_ref, w1s_ref, nb1_ref,
               nw2_ref, nb2_ref, ng_ref, nbeta_ref, out_ref):
    x = nodes_ref[...]
    a_s = as_ref[0] + as_ref[1]
    a_r = ar_ref[0] + ar_ref[1]
    ss = a_s[:, DE:DE + 1]
    sr = a_r[:, DE:DE + 1]
    sent = jnp.where(ss > 0, a_s[:, :DE] / jnp.where(ss > 0, ss, 1.0), 0.0)
    recv = jnp.where(sr > 0, a_r[:, :DE] / jnp.where(sr > 0, sr, 1.0), 0.0)
    h = (jnp.dot(x, w1n_ref[...], preferred_element_type=F32)
         + jnp.dot(recv, w1r_ref[...], preferred_element_type=F32)
         + jnp.dot(sent, w1s_ref[...], preferred_element_type=F32)
         + nb1_ref[...])
    h = jnp.maximum(h, 0.0)
    o = jnp.dot(h, nw2_ref[...], preferred_element_type=F32) + nb2_ref[...]
    mu = jnp.mean(o, axis=-1, keepdims=True)
    var = jnp.mean((o - mu) ** 2, axis=-1, keepdims=True)
    out_ref[...] = x + ((o - mu) * lax.rsqrt(var + 1e-5) * ng_ref[...]
                        + nbeta_ref[...])


_node = pl.pallas_call(
    _node_body,
    grid=(N // BN,),
    in_specs=[
        pl.BlockSpec((BN, DN), lambda i: (i, 0)),
        pl.BlockSpec((NC, BN, VW), lambda i: (0, i, 0)),
        pl.BlockSpec((NC, BN, VW), lambda i: (0, i + N // BN, 0)),
        pl.BlockSpec((DN, DN), lambda i: (0, 0)),
        pl.BlockSpec((DE, DN), lambda i: (0, 0)),
        pl.BlockSpec((DE, DN), lambda i: (0, 0)),
        pl.BlockSpec((1, DN), lambda i: (0, 0)),
        pl.BlockSpec((DN, DN), lambda i: (0, 0)),
        pl.BlockSpec((1, DN), lambda i: (0, 0)),
        pl.BlockSpec((1, DN), lambda i: (0, 0)),
        pl.BlockSpec((1, DN), lambda i: (0, 0)),
    ],
    out_specs=pl.BlockSpec((BN, DN), lambda i: (i, 0)),
    out_shape=jax.ShapeDtypeStruct((N, DN), F32),
)


def kernel(nodes, edges, senders, receivers,
           eW1, eb1, eW2, eb2, eg, ebeta,
           nW1, nb1, nW2, nb2, ng, nbeta,
           rW, rb, sW, sb):
    we = eW1[:DE]
    ws = eW1[DE:DE + DN]
    wr = eW1[DE + DN:]

    ps, pr = _proj(nodes, ws, wr)

    gs, gr = _gather(ps, pr, senders, receivers)

    wsr = jnp.concatenate([sW, rW], axis=1)
    bsr = jnp.concatenate([sb, rb]).reshape(1, 2)
    edges_out, v = _edge(edges, gs, gr, we, eb1.reshape(1, HID), eW2,
                         eb2.reshape(1, DE), eg.reshape(1, DE),
                         ebeta.reshape(1, DE), wsr, bsr)

    vflat = v.reshape(2 * E, VW)
    idx_comb = jnp.concatenate([senders, receivers + N]).reshape(S_NROWS, SROW)
    a = _scatter(vflat, idx_comb)

    nodes_out = _node(nodes, a, a, nW1[:DN], nW1[DN:DN + DE], nW1[DN + DE:],
                      nb1.reshape(1, HID), nW2, nb2.reshape(1, DN),
                      ng.reshape(1, DN), nbeta.reshape(1, DN))
    return nodes_out, edges_out


# speedup vs baseline: 9.7410x; 1.0208x over previous
import jax
import jax.numpy as jnp
from jax import lax
from jax.experimental import pallas as pl
from jax.experimental.pallas import tpu as pltpu
from jax.experimental.pallas import tpu_sc as plsc

N = 10000
E = 320000
DN = 128
DE = 16
HID = 128

NC, NS = 2, 16
NW = NC * NS

F32 = jnp.float32

_MESH = plsc.VectorSubcoreMesh(
    core_axis_name="c", subcore_axis_name="s", num_cores=NC, num_subcores=NS)

GROUP = 80
GCHUNK = 5
GB = GROUP * GCHUNK
G_EDGES_T = E // NW
G_ITERS = G_EDGES_T // GB


def _gather_body(ps_hbm, pr_hbm, sidx_hbm, ridx_hbm, gs_hbm, gr_hbm,
                 idx_v, buf_v, sem):
    cid = lax.axis_index("c")
    sid = lax.axis_index("s")
    wid = sid * NC + cid
    e0 = wid * G_EDGES_T

    def chunk(c, carry):
        base = e0 + c * GB
        pltpu.sync_copy(sidx_hbm.at[pl.ds(base, GB)], idx_v)
        descs = [
            pltpu.async_copy(ps_hbm.at[idx_v.at[pl.ds(j * GROUP, GROUP)]],
                             buf_v.at[pl.ds(j * GROUP, GROUP)], sem)
            for j in range(GCHUNK)
        ]
        for d in descs:
            d.wait()
        pltpu.sync_copy(buf_v, gs_hbm.at[pl.ds(base, GB)])

        pltpu.sync_copy(ridx_hbm.at[pl.ds(base, GB)], idx_v)
        descs = [
            pltpu.async_copy(pr_hbm.at[idx_v.at[pl.ds(j * GROUP, GROUP)]],
                             buf_v.at[pl.ds(j * GROUP, GROUP)], sem)
            for j in range(GCHUNK)
        ]
        for d in descs:
            d.wait()
        pltpu.sync_copy(buf_v, gr_hbm.at[pl.ds(base, GB)])
        return carry

    lax.fori_loop(0, G_ITERS, chunk, 0)


_gather = pl.kernel(
    _gather_body,
    out_type=(jax.ShapeDtypeStruct((E, DN), F32),
              jax.ShapeDtypeStruct((E, DN), F32)),
    mesh=_MESH,
    scratch_types=[
        pltpu.VMEM((GB,), jnp.int32),
        pltpu.VMEM((GB, DN), F32),
        pltpu.SemaphoreType.DMA,
    ],
)

VW = 32
SROW = 128
S_NROWS = E // SROW
S_BASE = S_NROWS // NW
S_XTRA = S_NROWS - S_BASE * NW
ASTRIPE = 2 * N // NS


def _scatter_body(vs_hbm, vr_hbm, s_hbm, r_hbm, out_hbm,
                  idx_v, val_v, stripe_v, a_sh, sem):
    cid = lax.axis_index("c")
    sid = lax.axis_index("s")
    wid = sid * NC + cid

    z = jnp.zeros((16,), F32)

    def zrow(i, carry):
        stripe_v[i, pl.ds(0, 16)] = z
        stripe_v[i, pl.ds(16, 16)] = z
        return carry

    lax.fori_loop(0, ASTRIPE, zrow, 0)
    pltpu.sync_copy(stripe_v, a_sh.at[pl.ds(sid * ASTRIPE, ASTRIPE)])
    plsc.subcore_barrier()

    row0 = S_BASE * wid + jnp.minimum(wid, S_XTRA)
    nrows = S_BASE + (wid < S_XTRA).astype(jnp.int32)

    def s_chunk(c, carry):
        r = row0 + c
        pltpu.sync_copy(s_hbm.at[pl.ds(r * SROW, SROW)], idx_v)
        pltpu.sync_copy(vs_hbm.at[pl.ds(r * SROW, SROW)], val_v)
        pltpu.sync_copy(val_v, a_sh.at[idx_v], add=True)
        return carry

    lax.fori_loop(0, nrows, s_chunk, 0)

    nvec = jnp.full((16,), N, jnp.int32)

    def r_chunk(c, carry):
        r = row0 + c
        pltpu.sync_copy(r_hbm.at[pl.ds(r * SROW, SROW)], idx_v)
        pltpu.sync_copy(vr_hbm.at[pl.ds(r * SROW, SROW)], val_v)
        for k in range(SROW // 16):
            idx_v[pl.ds(k * 16, 16)] = idx_v[pl.ds(k * 16, 16)] + nvec
        pltpu.sync_copy(val_v, a_sh.at[idx_v], add=True)
        return carry

    lax.fori_loop(0, nrows, r_chunk, 0)
    plsc.subcore_barrier()

    pltpu.sync_copy(a_sh.at[pl.ds(sid * ASTRIPE, ASTRIPE)], stripe_v)
    pltpu.sync_copy(stripe_v, out_hbm.at[cid, pl.ds(sid * ASTRIPE, ASTRIPE)])


_scatter = pl.kernel(
    _scatter_body,
    out_type=jax.ShapeDtypeStruct((NC, 2 * N, VW), F32),
    mesh=_MESH,
    scratch_types=[
        pltpu.VMEM((SROW,), jnp.int32),
        pltpu.VMEM((SROW, VW), F32),
        pltpu.VMEM((ASTRIPE, VW), F32),
        pltpu.VMEM_SHARED((2 * N, VW), F32),
        pltpu.SemaphoreType.DMA,
    ],
    compiler_params=pltpu.CompilerParams(use_tc_tiling_on_sc=False),
)

BN1 = 2000


def _proj_body(nodes_ref, ws_ref, wr_ref, ps_ref, pr_ref):
    x = nodes_ref[...]
    ps_ref[...] = jnp.dot(x, ws_ref[...], preferred_element_type=F32)
    pr_ref[...] = jnp.dot(x, wr_ref[...], preferred_element_type=F32)


_proj = pl.pallas_call(
    _proj_body,
    grid=(N // BN1,),
    in_specs=[
        pl.BlockSpec((BN1, DN), lambda i: (i, 0)),
        pl.BlockSpec((DN, DN), lambda i: (0, 0)),
        pl.BlockSpec((DN, DN), lambda i: (0, 0)),
    ],
    out_specs=(pl.BlockSpec((BN1, DN), lambda i: (i, 0)),
               pl.BlockSpec((BN1, DN), lambda i: (i, 0))),
    out_shape=(jax.ShapeDtypeStruct((N, DN), F32),
               jax.ShapeDtypeStruct((N, DN), F32)),
)

BE = 2000


def _edge_body(edges_ref, gs_ref, gr_ref, we_ref, eb1_ref, ew2_ref, eb2_ref,
               eg_ref, ebeta_ref, wsr_ref, bsr_ref, eo_ref, vs_ref, vr_ref):
    ed = edges_ref[...]
    h = (jnp.dot(ed, we_ref[...], preferred_element_type=F32)
         + gs_ref[...] + gr_ref[...] + eb1_ref[...])
    h = jnp.maximum(h, 0.0)
    o = jnp.dot(h, ew2_ref[...], preferred_element_type=F32) + eb2_ref[...]
    mu = jnp.mean(o, axis=-1, keepdims=True)
    var = jnp.mean((o - mu) ** 2, axis=-1, keepdims=True)
    u = (o - mu) * lax.rsqrt(var + 1e-5) * eg_ref[...] + ebeta_ref[...]
    eo_ref[...] = ed + u
    lg = jnp.dot(ed, wsr_ref[...], preferred_element_type=F32) + bsr_ref[...]
    ee = jnp.exp(lg)
    es = ee[:, 0:1]
    er = ee[:, 1:2]
    pad = jnp.zeros((BE, VW - DE - 1), F32)
    vs_ref[...] = jnp.concatenate([u * es, es, pad], axis=1)
    vr_ref[...] = jnp.concatenate([u * er, er, pad], axis=1)


_edge = pl.pallas_call(
    _edge_body,
    grid=(E // BE,),
    in_specs=[
        pl.BlockSpec((BE, DE), lambda i: (i, 0)),
        pl.BlockSpec((BE, DN), lambda i: (i, 0)),
        pl.BlockSpec((BE, DN), lambda i: (i, 0)),
        pl.BlockSpec((DE, DN), lambda i: (0, 0)),
        pl.BlockSpec((1, DN), lambda i: (0, 0)),
        pl.BlockSpec((DN, DE), lambda i: (0, 0)),
        pl.BlockSpec((1, DE), lambda i: (0, 0)),
        pl.BlockSpec((1, DE), lambda i: (0, 0)),
        pl.BlockSpec((1, DE), lambda i: (0, 0)),
        pl.BlockSpec((DE, 2), lambda i: (0, 0)),
        pl.BlockSpec((1, 2), lambda i: (0, 0)),
    ],
    out_specs=(pl.BlockSpec((BE, DE), lambda i: (i, 0)),
               pl.BlockSpec((BE, VW), lambda i: (i, 0)),
               pl.BlockSpec((BE, VW), lambda i: (i, 0))),
    out_shape=(jax.ShapeDtypeStruct((E, DE), F32),
               jax.ShapeDtypeStruct((E, VW), F32),
               jax.ShapeDtypeStruct((E, VW), F32)),
)

BN = 2000


def _node_body(nodes_ref, as_ref, ar_ref, w1n_ref, w1r_ref, w1s_ref, nb1_ref,
               nw2_ref, nb2_ref, ng_ref, nbeta_ref, out_ref):
    x = nodes_ref[...]
    a_s = as_ref[0] + as_ref[1]
    a_r = ar_ref[0] + ar_ref[1]
    ss = a_s[:, DE:DE + 1]
    sr = a_r[:, DE:DE + 1]
    sent = jnp.where(ss > 0, a_s[:, :DE] / jnp.where(ss > 0, ss, 1.0), 0.0)
    recv = jnp.where(sr > 0, a_r[:, :DE] / jnp.where(sr > 0, sr, 1.0), 0.0)
    h = (jnp.dot(x, w1n_ref[...], preferred_element_type=F32)
         + jnp.dot(recv, w1r_ref[...], preferred_element_type=F32)
         + jnp.dot(sent, w1s_ref[...], preferred_element_type=F32)
         + nb1_ref[...])
    h = jnp.maximum(h, 0.0)
    o = jnp.dot(h, nw2_ref[...], preferred_element_type=F32) + nb2_ref[...]
    mu = jnp.mean(o, axis=-1, keepdims=True)
    var = jnp.mean((o - mu) ** 2, axis=-1, keepdims=True)
    out_ref[...] = x + ((o - mu) * lax.rsqrt(var + 1e-5) * ng_ref[...]
                        + nbeta_ref[...])


_node = pl.pallas_call(
    _node_body,
    grid=(N // BN,),
    in_specs=[
        pl.BlockSpec((BN, DN), lambda i: (i, 0)),
        pl.BlockSpec((NC, BN, VW), lambda i: (0, i, 0)),
        pl.BlockSpec((NC, BN, VW), lambda i: (0, i + N // BN, 0)),
        pl.BlockSpec((DN, DN), lambda i: (0, 0)),
        pl.BlockSpec((DE, DN), lambda i: (0, 0)),
        pl.BlockSpec((DE, DN), lambda i: (0, 0)),
        pl.BlockSpec((1, DN), lambda i: (0, 0)),
        pl.BlockSpec((DN, DN), lambda i: (0, 0)),
        pl.BlockSpec((1, DN), lambda i: (0, 0)),
        pl.BlockSpec((1, DN), lambda i: (0, 0)),
        pl.BlockSpec((1, DN), lambda i: (0, 0)),
    ],
    out_specs=pl.BlockSpec((BN, DN), lambda i: (i, 0)),
    out_shape=jax.ShapeDtypeStruct((N, DN), F32),
)


def kernel(nodes, edges, senders, receivers,
           eW1, eb1, eW2, eb2, eg, ebeta,
           nW1, nb1, nW2, nb2, ng, nbeta,
           rW, rb, sW, sb):
    we = eW1[:DE]
    ws = eW1[DE:DE + DN]
    wr = eW1[DE + DN:]

    ps, pr = _proj(nodes, ws, wr)

    gs, gr = _gather(ps, pr, senders, receivers)

    wsr = jnp.concatenate([sW, rW], axis=1)
    bsr = jnp.concatenate([sb, rb]).reshape(1, 2)
    edges_out, vs, vr = _edge(edges, gs, gr, we, eb1.reshape(1, HID), eW2,
                              eb2.reshape(1, DE), eg.reshape(1, DE),
                              ebeta.reshape(1, DE), wsr, bsr)

    a = _scatter(vs, vr, senders, receivers)

    nodes_out = _node(nodes, a, a, nW1[:DN], nW1[DN:DN + DE], nW1[DN + DE:],
                      nb1.reshape(1, HID), nW2, nb2.reshape(1, DN),
                      ng.reshape(1, DN), nbeta.reshape(1, DN))
    return nodes_out, edges_out


# speedup vs baseline: 10.8926x; 1.1182x over previous
import jax
import jax.numpy as jnp
from jax import lax
from jax.experimental import pallas as pl
from jax.experimental.pallas import tpu as pltpu
from jax.experimental.pallas import tpu_sc as plsc

N = 10000
E = 320000
DN = 128
DE = 16
HID = 128

NC, NS = 2, 16
NW = NC * NS

F32 = jnp.float32

H0 = 166400
H1 = E - H0
BE = 3200

_MESH = plsc.VectorSubcoreMesh(
    core_axis_name="c", subcore_axis_name="s", num_cores=NC, num_subcores=NS)

GROUP = 80
GCHUNK = 5
GB = GROUP * GCHUNK


def _make_gather(e_lo, e_cnt):
    edges_t = e_cnt // NW
    iters = edges_t // GB

    def body(ps_hbm, pr_hbm, sidx_hbm, ridx_hbm, gs_hbm, gr_hbm,
             idx_v, buf_v, sem):
        cid = lax.axis_index("c")
        sid = lax.axis_index("s")
        wid = sid * NC + cid
        e0 = wid * edges_t

        def chunk(c, carry):
            src = e_lo + e0 + c * GB
            dst = e0 + c * GB
            pltpu.sync_copy(sidx_hbm.at[pl.ds(src, GB)], idx_v)
            descs = [
                pltpu.async_copy(ps_hbm.at[idx_v.at[pl.ds(j * GROUP, GROUP)]],
                                 buf_v.at[pl.ds(j * GROUP, GROUP)], sem)
                for j in range(GCHUNK)
            ]
            for d in descs:
                d.wait()
            pltpu.sync_copy(buf_v, gs_hbm.at[pl.ds(dst, GB)])

            pltpu.sync_copy(ridx_hbm.at[pl.ds(src, GB)], idx_v)
            descs = [
                pltpu.async_copy(pr_hbm.at[idx_v.at[pl.ds(j * GROUP, GROUP)]],
                                 buf_v.at[pl.ds(j * GROUP, GROUP)], sem)
                for j in range(GCHUNK)
            ]
            for d in descs:
                d.wait()
            pltpu.sync_copy(buf_v, gr_hbm.at[pl.ds(dst, GB)])
            return carry

        lax.fori_loop(0, iters, chunk, 0)

    return pl.kernel(
        body,
        out_type=(jax.ShapeDtypeStruct((e_cnt, DN), F32),
                  jax.ShapeDtypeStruct((e_cnt, DN), F32)),
        mesh=_MESH,
        scratch_types=[
            pltpu.VMEM((GB,), jnp.int32),
            pltpu.VMEM((GB, DN), F32),
            pltpu.SemaphoreType.DMA,
        ],
    )


_gather0 = _make_gather(0, H0)
_gather1 = _make_gather(H0, H1)

VW = 32
SROW = 128
ASTRIPE = 2 * N // NS


def _make_scatter(g_lo, v_cnt):
    n_groups = v_cnt // SROW
    s_base = n_groups // NW
    s_xtra = n_groups - s_base * NW

    def body(v_hbm, ii_hbm, out_hbm, idx_v, val_v, stripe_v, a_sh, sem):
        cid = lax.axis_index("c")
        sid = lax.axis_index("s")
        wid = sid * NC + cid

        z = jnp.zeros((16,), F32)

        def zrow(i, carry):
            stripe_v[i, pl.ds(0, 16)] = z
            stripe_v[i, pl.ds(16, 16)] = z
            return carry

        lax.fori_loop(0, ASTRIPE, zrow, 0)
        pltpu.sync_copy(stripe_v, a_sh.at[pl.ds(sid * ASTRIPE, ASTRIPE)])
        plsc.subcore_barrier()

        row0 = s_base * wid + jnp.minimum(wid, s_xtra)
        nrows = s_base + (wid < s_xtra).astype(jnp.int32)

        def chunk(c, carry):
            g = row0 + c
            pltpu.sync_copy(ii_hbm.at[pl.ds((g_lo + g) * SROW, SROW)], idx_v)
            pltpu.sync_copy(v_hbm.at[pl.ds(g * SROW, SROW)], val_v)
            pltpu.sync_copy(val_v, a_sh.at[idx_v], add=True)
            return carry

        lax.fori_loop(0, nrows, chunk, 0)
        plsc.subcore_barrier()

        pltpu.sync_copy(a_sh.at[pl.ds(sid * ASTRIPE, ASTRIPE)], stripe_v)
        pltpu.sync_copy(stripe_v, out_hbm.at[cid, pl.ds(sid * ASTRIPE, ASTRIPE)])

    return pl.kernel(
        body,
        out_type=jax.ShapeDtypeStruct((NC, 2 * N, VW), F32),
        mesh=_MESH,
        scratch_types=[
            pltpu.VMEM((SROW,), jnp.int32),
            pltpu.VMEM((SROW, VW), F32),
            pltpu.VMEM((ASTRIPE, VW), F32),
            pltpu.VMEM_SHARED((2 * N, VW), F32),
            pltpu.SemaphoreType.DMA,
        ],
        compiler_params=pltpu.CompilerParams(use_tc_tiling_on_sc=False),
    )


_scatter0 = _make_scatter(0, 2 * H0)
_scatter1 = _make_scatter(2 * H0 // SROW, 2 * H1)

BN1 = 2000


def _proj_body(nodes_ref, ws_ref, wr_ref, ps_ref, pr_ref):
    x = nodes_ref[...]
    ps_ref[...] = jnp.dot(x, ws_ref[...], preferred_element_type=F32)
    pr_ref[...] = jnp.dot(x, wr_ref[...], preferred_element_type=F32)


_proj = pl.pallas_call(
    _proj_body,
    grid=(N // BN1,),
    in_specs=[
        pl.BlockSpec((BN1, DN), lambda i: (i, 0)),
        pl.BlockSpec((DN, DN), lambda i: (0, 0)),
        pl.BlockSpec((DN, DN), lambda i: (0, 0)),
    ],
    out_specs=(pl.BlockSpec((BN1, DN), lambda i: (i, 0)),
               pl.BlockSpec((BN1, DN), lambda i: (i, 0))),
    out_shape=(jax.ShapeDtypeStruct((N, DN), F32),
               jax.ShapeDtypeStruct((N, DN), F32)),
)


def _edge_body(edges_ref, gs_ref, gr_ref, we_ref, eb1_ref, ew2_ref, eb2_ref,
               eg_ref, ebeta_ref, wsr_ref, bsr_ref, eo_ref, v_ref):
    ed = edges_ref[...]
    h = (jnp.dot(ed, we_ref[...], preferred_element_type=F32)
         + gs_ref[...] + gr_ref[...] + eb1_ref[...])
    h = jnp.maximum(h, 0.0)
    o = jnp.dot(h, ew2_ref[...], preferred_element_type=F32) + eb2_ref[...]
    mu = jnp.mean(o, axis=-1, keepdims=True)
    var = jnp.mean((o - mu) ** 2, axis=-1, keepdims=True)
    u = (o - mu) * lax.rsqrt(var + 1e-5) * eg_ref[...] + ebeta_ref[...]
    eo_ref[...] = ed + u
    lg = jnp.dot(ed, wsr_ref[...], preferred_element_type=F32) + bsr_ref[...]
    ee = jnp.exp(lg)
    es = ee[:, 0:1]
    er = ee[:, 1:2]
    pad = jnp.zeros((BE, VW - DE - 1), F32)
    v_ref[...] = jnp.concatenate([u * es, es, pad, u * er, er, pad], axis=1)


def _make_edge(b_lo, e_cnt):
    return pl.pallas_call(
        _edge_body,
        grid=(e_cnt // BE,),
        in_specs=[
            pl.BlockSpec((BE, DE), lambda i: (i + b_lo, 0)),
            pl.BlockSpec((BE, DN), lambda i: (i, 0)),
            pl.BlockSpec((BE, DN), lambda i: (i, 0)),
            pl.BlockSpec((DE, DN), lambda i: (0, 0)),
            pl.BlockSpec((1, DN), lambda i: (0, 0)),
            pl.BlockSpec((DN, DE), lambda i: (0, 0)),
            pl.BlockSpec((1, DE), lambda i: (0, 0)),
            pl.BlockSpec((1, DE), lambda i: (0, 0)),
            pl.BlockSpec((1, DE), lambda i: (0, 0)),
            pl.BlockSpec((DE, 2), lambda i: (0, 0)),
            pl.BlockSpec((1, 2), lambda i: (0, 0)),
        ],
        out_specs=(pl.BlockSpec((BE, DE), lambda i: (i, 0)),
                   pl.BlockSpec((BE, 2 * VW), lambda i: (i, 0))),
        out_shape=(jax.ShapeDtypeStruct((e_cnt, DE), F32),
                   jax.ShapeDtypeStruct((e_cnt, 2 * VW), F32)),
    )


_edge0 = _make_edge(0, H0)
_edge1 = _make_edge(H0 // BE, H1)

BN = 2000


def _node_body(nodes_ref, as0_ref, ar0_ref, as1_ref, ar1_ref,
               w1n_ref, w1r_ref, w1s_ref, nb1_ref,
               nw2_ref, nb2_ref, ng_ref, nbeta_ref, out_ref):
    x = nodes_ref[...]
    a_s = as0_ref[0] + as0_ref[1] + as1_ref[0] + as1_ref[1]
    a_r = ar0_ref[0] + ar0_ref[1] + ar1_ref[0] + ar1_ref[1]
    ss = a_s[:, DE:DE + 1]
    sr = a_r[:, DE:DE + 1]
    sent = jnp.where(ss > 0, a_s[:, :DE] / jnp.where(ss > 0, ss, 1.0), 0.0)
    recv = jnp.where(sr > 0, a_r[:, :DE] / jnp.where(sr > 0, sr, 1.0), 0.0)
    h = (jnp.dot(x, w1n_ref[...], preferred_element_type=F32)
         + jnp.dot(recv, w1r_ref[...], preferred_element_type=F32)
         + jnp.dot(sent, w1s_ref[...], preferred_element_type=F32)
         + nb1_ref[...])
    h = jnp.maximum(h, 0.0)
    o = jnp.dot(h, nw2_ref[...], preferred_element_type=F32) + nb2_ref[...]
    mu = jnp.mean(o, axis=-1, keepdims=True)
    var = jnp.mean((o - mu) ** 2, axis=-1, keepdims=True)
    out_ref[...] = x + ((o - mu) * lax.rsqrt(var + 1e-5) * ng_ref[...]
                        + nbeta_ref[...])


_A_SPEC_S = pl.BlockSpec((NC, BN, VW), lambda i: (0, i, 0))
_A_SPEC_R = pl.BlockSpec((NC, BN, VW), lambda i: (0, i + N // BN, 0))

_node = pl.pallas_call(
    _node_body,
    grid=(N // BN,),
    in_specs=[
        pl.BlockSpec((BN, DN), lambda i: (i, 0)),
        _A_SPEC_S,
        _A_SPEC_R,
        _A_SPEC_S,
        _A_SPEC_R,
        pl.BlockSpec((DN, DN), lambda i: (0, 0)),
        pl.BlockSpec((DE, DN), lambda i: (0, 0)),
        pl.BlockSpec((DE, DN), lambda i: (0, 0)),
        pl.BlockSpec((1, DN), lambda i: (0, 0)),
        pl.BlockSpec((DN, DN), lambda i: (0, 0)),
        pl.BlockSpec((1, DN), lambda i: (0, 0)),
        pl.BlockSpec((1, DN), lambda i: (0, 0)),
        pl.BlockSpec((1, DN), lambda i: (0, 0)),
    ],
    out_specs=pl.BlockSpec((BN, DN), lambda i: (i, 0)),
    out_shape=jax.ShapeDtypeStruct((N, DN), F32),
)


def kernel(nodes, edges, senders, receivers,
           eW1, eb1, eW2, eb2, eg, ebeta,
           nW1, nb1, nW2, nb2, ng, nbeta,
           rW, rb, sW, sb):
    we = eW1[:DE]
    ws = eW1[DE:DE + DN]
    wr = eW1[DE + DN:]

    ps, pr = _proj(nodes, ws, wr)

    gs0, gr0 = _gather0(ps, pr, senders, receivers)
    gs1, gr1 = _gather1(ps, pr, senders, receivers)

    wsr = jnp.concatenate([sW, rW], axis=1)
    bsr = jnp.concatenate([sb, rb]).reshape(1, 2)
    ew = (we, eb1.reshape(1, HID), eW2, eb2.reshape(1, DE),
          eg.reshape(1, DE), ebeta.reshape(1, DE), wsr, bsr)
    eo0, v0 = _edge0(edges, gs0, gr0, *ew)
    eo1, v1 = _edge1(edges, gs1, gr1, *ew)

    ii = jnp.stack([senders, receivers + N], axis=1).reshape(2 * E)
    a0 = _scatter0(v0.reshape(2 * H0, VW), ii)
    a1 = _scatter1(v1.reshape(2 * H1, VW), ii)

    nodes_out = _node(nodes, a0, a0, a1, a1,
                      nW1[:DN], nW1[DN:DN + DE], nW1[DN + DE:],
                      nb1.reshape(1, HID), nW2, nb2.reshape(1, DN),
                      ng.reshape(1, DN), nbeta.reshape(1, DN))
    edges_out = jnp.concatenate([eo0, eo1])
    return nodes_out, edges_out


# speedup vs baseline: 11.0203x; 1.0117x over previous
import jax
import jax.numpy as jnp
from jax import lax
from jax.experimental import pallas as pl
from jax.experimental.pallas import tpu as pltpu
from jax.experimental.pallas import tpu_sc as plsc

N = 10000
E = 320000
DN = 128
DE = 16
HID = 128

NC, NS = 2, 16
NW = NC * NS

F32 = jnp.float32

H0 = 166400
H1 = E - H0
BE = 3200

_MESH = plsc.VectorSubcoreMesh(
    core_axis_name="c", subcore_axis_name="s", num_cores=NC, num_subcores=NS)

GROUP = 80
GCHUNK = 5
GB = GROUP * GCHUNK


def _make_gather(e_lo, e_cnt):
    edges_t = e_cnt // NW
    iters = edges_t // GB

    def body(ps_hbm, pr_hbm, sidx_hbm, ridx_hbm, gs_hbm, gr_hbm,
             idx_v, buf_v, sem):
        cid = lax.axis_index("c")
        sid = lax.axis_index("s")
        wid = sid * NC + cid
        e0 = wid * edges_t

        def chunk(c, carry):
            src = e_lo + e0 + c * GB
            dst = e0 + c * GB
            pltpu.sync_copy(sidx_hbm.at[pl.ds(src, GB)], idx_v)
            descs = [
                pltpu.async_copy(ps_hbm.at[idx_v.at[pl.ds(j * GROUP, GROUP)]],
                                 buf_v.at[pl.ds(j * GROUP, GROUP)], sem)
                for j in range(GCHUNK)
            ]
            for d in descs:
                d.wait()
            pltpu.sync_copy(buf_v, gs_hbm.at[pl.ds(dst, GB)])

            pltpu.sync_copy(ridx_hbm.at[pl.ds(src, GB)], idx_v)
            descs = [
                pltpu.async_copy(pr_hbm.at[idx_v.at[pl.ds(j * GROUP, GROUP)]],
                                 buf_v.at[pl.ds(j * GROUP, GROUP)], sem)
                for j in range(GCHUNK)
            ]
            for d in descs:
                d.wait()
            pltpu.sync_copy(buf_v, gr_hbm.at[pl.ds(dst, GB)])
            return carry

        lax.fori_loop(0, iters, chunk, 0)

    return pl.kernel(
        body,
        out_type=(jax.ShapeDtypeStruct((e_cnt, DN), F32),
                  jax.ShapeDtypeStruct((e_cnt, DN), F32)),
        mesh=_MESH,
        scratch_types=[
            pltpu.VMEM((GB,), jnp.int32),
            pltpu.VMEM((GB, DN), F32),
            pltpu.SemaphoreType.DMA,
        ],
    )


_gather0 = _make_gather(0, H0)
_gather1 = _make_gather(H0, H1)

VW = 32
SROW = 128
ASTRIPE = 2 * N // NS


def _make_scatter(g_lo, v_cnt):
    n_groups = v_cnt // SROW
    s_base = n_groups // NW
    s_xtra = n_groups - s_base * NW

    def body(v_hbm, ii_hbm, out_hbm, idx_v, val_v, stripe_v, a_sh, sem):
        cid = lax.axis_index("c")
        sid = lax.axis_index("s")
        wid = sid * NC + cid

        z = jnp.zeros((16,), F32)

        def zrow(i, carry):
            stripe_v[i, pl.ds(0, 16)] = z
            stripe_v[i, pl.ds(16, 16)] = z
            return carry

        lax.fori_loop(0, ASTRIPE, zrow, 0)
        pltpu.sync_copy(stripe_v, a_sh.at[pl.ds(sid * ASTRIPE, ASTRIPE)])
        plsc.subcore_barrier()

        row0 = s_base * wid + jnp.minimum(wid, s_xtra)
        nrows = s_base + (wid < s_xtra).astype(jnp.int32)

        def chunk(c, carry):
            slot = lax.rem(c, 2)
            g = row0 + c
            pltpu.sync_copy(ii_hbm.at[pl.ds((g_lo + g) * SROW, SROW)],
                            idx_v.at[slot])
            pltpu.sync_copy(v_hbm.at[pl.ds(g * SROW, SROW)], val_v.at[slot])

            @pl.when(c > 0)
            def _():
                pltpu.make_async_copy(val_v.at[slot],
                                      a_sh.at[idx_v.at[slot]], sem).wait()

            pltpu.async_copy(val_v.at[slot], a_sh.at[idx_v.at[slot]], sem,
                             add=True)
            return carry

        lax.fori_loop(0, nrows, chunk, 0)

        @pl.when(nrows > 0)
        def _():
            pltpu.make_async_copy(val_v.at[0], a_sh.at[idx_v.at[0]],
                                  sem).wait()

        plsc.subcore_barrier()

        pltpu.sync_copy(a_sh.at[pl.ds(sid * ASTRIPE, ASTRIPE)], stripe_v)
        pltpu.sync_copy(stripe_v, out_hbm.at[cid, pl.ds(sid * ASTRIPE, ASTRIPE)])

    return pl.kernel(
        body,
        out_type=jax.ShapeDtypeStruct((NC, 2 * N, VW), F32),
        mesh=_MESH,
        scratch_types=[
            pltpu.VMEM((2, SROW), jnp.int32),
            pltpu.VMEM((2, SROW, VW), F32),
            pltpu.VMEM((ASTRIPE, VW), F32),
            pltpu.VMEM_SHARED((2 * N, VW), F32),
            pltpu.SemaphoreType.DMA,
        ],
        compiler_params=pltpu.CompilerParams(use_tc_tiling_on_sc=False),
    )


_scatter0 = _make_scatter(0, 2 * H0)
_scatter1 = _make_scatter(2 * H0 // SROW, 2 * H1)

BN1 = 2000


def _proj_body(nodes_ref, ws_ref, wr_ref, ps_ref, pr_ref):
    x = nodes_ref[...]
    ps_ref[...] = jnp.dot(x, ws_ref[...], preferred_element_type=F32)
    pr_ref[...] = jnp.dot(x, wr_ref[...], preferred_element_type=F32)


_proj = pl.pallas_call(
    _proj_body,
    grid=(N // BN1,),
    in_specs=[
        pl.BlockSpec((BN1, DN), lambda i: (i, 0)),
        pl.BlockSpec((DN, DN), lambda i: (0, 0)),
        pl.BlockSpec((DN, DN), lambda i: (0, 0)),
    ],
    out_specs=(pl.BlockSpec((BN1, DN), lambda i: (i, 0)),
               pl.BlockSpec((BN1, DN), lambda i: (i, 0))),
    out_shape=(jax.ShapeDtypeStruct((N, DN), F32),
               jax.ShapeDtypeStruct((N, DN), F32)),
)


def _edge_body(edges_ref, gs_ref, gr_ref, we_ref, eb1_ref, ew2_ref, eb2_ref,
               eg_ref, ebeta_ref, wsr_ref, bsr_ref, eo_ref, v_ref):
    ed = edges_ref[...]
    h = (jnp.dot(ed, we_ref[...], preferred_element_type=F32)
         + gs_ref[...] + gr_ref[...] + eb1_ref[...])
    h = jnp.maximum(h, 0.0)
    o = jnp.dot(h, ew2_ref[...], preferred_element_type=F32) + eb2_ref[...]
    mu = jnp.mean(o, axis=-1, keepdims=True)
    var = jnp.mean((o - mu) ** 2, axis=-1, keepdims=True)
    u = (o - mu) * lax.rsqrt(var + 1e-5) * eg_ref[...] + ebeta_ref[...]
    eo_ref[...] = ed + u
    lg = jnp.dot(ed, wsr_ref[...], preferred_element_type=F32) + bsr_ref[...]
    ee = jnp.exp(lg)
    es = ee[:, 0:1]
    er = ee[:, 1:2]
    pad = jnp.zeros((BE, VW - DE - 1), F32)
    v_ref[...] = jnp.concatenate([u * es, es, pad, u * er, er, pad], axis=1)


def _make_edge(b_lo, e_cnt):
    return pl.pallas_call(
        _edge_body,
        grid=(e_cnt // BE,),
        in_specs=[
            pl.BlockSpec((BE, DE), lambda i: (i + b_lo, 0)),
            pl.BlockSpec((BE, DN), lambda i: (i, 0)),
            pl.BlockSpec((BE, DN), lambda i: (i, 0)),
            pl.BlockSpec((DE, DN), lambda i: (0, 0)),
            pl.BlockSpec((1, DN), lambda i: (0, 0)),
            pl.BlockSpec((DN, DE), lambda i: (0, 0)),
            pl.BlockSpec((1, DE), lambda i: (0, 0)),
            pl.BlockSpec((1, DE), lambda i: (0, 0)),
            pl.BlockSpec((1, DE), lambda i: (0, 0)),
            pl.BlockSpec((DE, 2), lambda i: (0, 0)),
            pl.BlockSpec((1, 2), lambda i: (0, 0)),
        ],
        out_specs=(pl.BlockSpec((BE, DE), lambda i: (i, 0)),
                   pl.BlockSpec((BE, 2 * VW), lambda i: (i, 0))),
        out_shape=(jax.ShapeDtypeStruct((e_cnt, DE), F32),
                   jax.ShapeDtypeStruct((e_cnt, 2 * VW), F32)),
    )


_edge0 = _make_edge(0, H0)
_edge1 = _make_edge(H0 // BE, H1)

BN = 2000


def _node_body(nodes_ref, as0_ref, ar0_ref, as1_ref, ar1_ref,
               w1n_ref, w1r_ref, w1s_ref, nb1_ref,
               nw2_ref, nb2_ref, ng_ref, nbeta_ref, out_ref):
    x = nodes_ref[...]
    a_s = as0_ref[0] + as0_ref[1] + as1_ref[0] + as1_ref[1]
    a_r = ar0_ref[0] + ar0_ref[1] + ar1_ref[0] + ar1_ref[1]
    ss = a_s[:, DE:DE + 1]
    sr = a_r[:, DE:DE + 1]
    sent = jnp.where(ss > 0, a_s[:, :DE] / jnp.where(ss > 0, ss, 1.0), 0.0)
    recv = jnp.where(sr > 0, a_r[:, :DE] / jnp.where(sr > 0, sr, 1.0), 0.0)
    h = (jnp.dot(x, w1n_ref[...], preferred_element_type=F32)
         + jnp.dot(recv, w1r_ref[...], preferred_element_type=F32)
         + jnp.dot(sent, w1s_ref[...], preferred_element_type=F32)
         + nb1_ref[...])
    h = jnp.maximum(h, 0.0)
    o = jnp.dot(h, nw2_ref[...], preferred_element_type=F32) + nb2_ref[...]
    mu = jnp.mean(o, axis=-1, keepdims=True)
    var = jnp.mean((o - mu) ** 2, axis=-1, keepdims=True)
    out_ref[...] = x + ((o - mu) * lax.rsqrt(var + 1e-5) * ng_ref[...]
                        + nbeta_ref[...])


_A_SPEC_S = pl.BlockSpec((NC, BN, VW), lambda i: (0, i, 0))
_A_SPEC_R = pl.BlockSpec((NC, BN, VW), lambda i: (0, i + N // BN, 0))

_node = pl.pallas_call(
    _node_body,
    grid=(N // BN,),
    in_specs=[
        pl.BlockSpec((BN, DN), lambda i: (i, 0)),
        _A_SPEC_S,
        _A_SPEC_R,
        _A_SPEC_S,
        _A_SPEC_R,
        pl.BlockSpec((DN, DN), lambda i: (0, 0)),
        pl.BlockSpec((DE, DN), lambda i: (0, 0)),
        pl.BlockSpec((DE, DN), lambda i: (0, 0)),
        pl.BlockSpec((1, DN), lambda i: (0, 0)),
        pl.BlockSpec((DN, DN), lambda i: (0, 0)),
        pl.BlockSpec((1, DN), lambda i: (0, 0)),
        pl.BlockSpec((1, DN), lambda i: (0, 0)),
        pl.BlockSpec((1, DN), lambda i: (0, 0)),
    ],
    out_specs=pl.BlockSpec((BN, DN), lambda i: (i, 0)),
    out_shape=jax.ShapeDtypeStruct((N, DN), F32),
)


def kernel(nodes, edges, senders, receivers,
           eW1, eb1, eW2, eb2, eg, ebeta,
           nW1, nb1, nW2, nb2, ng, nbeta,
           rW, rb, sW, sb):
    we = eW1[:DE]
    ws = eW1[DE:DE + DN]
    wr = eW1[DE + DN:]

    ps, pr = _proj(nodes, ws, wr)

    gs0, gr0 = _gather0(ps, pr, senders, receivers)
    gs1, gr1 = _gather1(ps, pr, senders, receivers)

    wsr = jnp.concatenate([sW, rW], axis=1)
    bsr = jnp.concatenate([sb, rb]).reshape(1, 2)
    ew = (we, eb1.reshape(1, HID), eW2, eb2.reshape(1, DE),
          eg.reshape(1, DE), ebeta.reshape(1, DE), wsr, bsr)
    eo0, v0 = _edge0(edges, gs0, gr0, *ew)
    eo1, v1 = _edge1(edges, gs1, gr1, *ew)

    ii = jnp.stack([senders, receivers + N], axis=1).reshape(2 * E)
    a0 = _scatter0(v0.reshape(2 * H0, VW), ii)
    a1 = _scatter1(v1.reshape(2 * H1, VW), ii)

    nodes_out = _node(nodes, a0, a0, a1, a1,
                      nW1[:DN], nW1[DN:DN + DE], nW1[DN + DE:],
                      nb1.reshape(1, HID), nW2, nb2.reshape(1, DN),
                      ng.reshape(1, DN), nbeta.reshape(1, DN))
    edges_out = jnp.concatenate([eo0, eo1])
    return nodes_out, edges_out
